# Initial kernel scaffold; baseline (speedup 1.0000x reference)
import jax
import jax.numpy as jnp
from jax.experimental import pallas as pl


def kernel(inputs, codebook):
    raise NotImplementedError("write your pallas kernel here")


# speedup vs baseline: 3.9196x; 3.9196x over previous
import jax
import jax.numpy as jnp
from jax.experimental import pallas as pl
from jax.experimental.pallas import tpu as pltpu

_B, _D, _N, _K = 32, 64, 1024, 1024


def _vq_body(x_ref, cb_ref, loss_ref, q_ref, ppl_ref, oh_ref, counts_ref):
    i = pl.program_id(0)
    x = x_ref[0]
    cb = cb_ref[...]

    cn = jnp.sum(cb * cb, axis=1, keepdims=True)
    scores = jax.lax.dot_general(
        cb, x, (((1,), (0,)), ((), ())), preferred_element_type=jnp.float32)
    dist = cn - 2.0 * scores

    minv = jnp.min(dist, axis=0, keepdims=True)
    iota = jax.lax.broadcasted_iota(jnp.int32, dist.shape, 0)
    idx = jnp.min(jnp.where(dist == minv, iota, _K), axis=0, keepdims=True)
    oh = (iota == idx).astype(jnp.float32)
    oh_ref[0] = oh

    q = jax.lax.dot_general(
        cb, oh, (((0,), (0,)), ((), ())), preferred_element_type=jnp.float32)
    q_ref[0] = q

    diff = q - x
    block_loss = jnp.sum(diff * diff)

    @pl.when(i == 0)
    def _init():
        loss_ref[0, 0] = 0.0
        counts_ref[...] = jnp.zeros_like(counts_ref)

    loss_ref[0, 0] += block_loss * (0.25 / (_B * _N * _D))
    counts_ref[...] += jnp.sum(oh, axis=1, keepdims=True)

    @pl.when(i == _B - 1)
    def _finalize():
        p = counts_ref[...] * (1.0 / (_B * _N))
        ent = jnp.sum(p * jnp.log(p + 1e-10))
        ppl_ref[0, 0] = jnp.exp(-ent)


def _vq_call(inputs, codebook, interpret=False):
    return pl.pallas_call(
        _vq_body,
        grid=(_B,),
        in_specs=[
            pl.BlockSpec((1, _D, _N), lambda i: (i, 0, 0)),
            pl.BlockSpec((_K, _D), lambda i: (0, 0)),
        ],
        out_specs=[
            pl.BlockSpec(memory_space=pltpu.SMEM),
            pl.BlockSpec((1, _D, _N), lambda i: (i, 0, 0)),
            pl.BlockSpec(memory_space=pltpu.SMEM),
            pl.BlockSpec((1, _K, _N), lambda i: (i, 0, 0)),
        ],
        out_shape=[
            jax.ShapeDtypeStruct((1, 1), jnp.float32),
            jax.ShapeDtypeStruct((_B, _D, _N), jnp.float32),
            jax.ShapeDtypeStruct((1, 1), jnp.float32),
            jax.ShapeDtypeStruct((_B, _K, _N), jnp.float32),
        ],
        scratch_shapes=[pltpu.VMEM((_K, 1), jnp.float32)],
        compiler_params=pltpu.CompilerParams(
            dimension_semantics=("arbitrary",)),
        interpret=interpret,
    )(inputs, codebook)


def kernel(inputs, codebook):
    loss, q, ppl, oh = _vq_call(inputs, codebook)
    return (loss[0, 0], q, ppl[0, 0], oh)


# speedup vs baseline: 4.6439x; 1.1848x over previous
import jax
import jax.numpy as jnp
from jax.experimental import pallas as pl
from jax.experimental.pallas import tpu as pltpu

_B, _D, _N, _K = 32, 64, 1024, 1024


def _vq_body(x_ref, cb_ref, loss_ref, q_ref, ppl_ref, oh_ref,
             cbn2_ref, cn_ref, iota_ref, lacc_ref, counts_ref):
    i = pl.program_id(0)
    cb = cb_ref[...]

    @pl.when(i == 0)
    def _init():
        cbn2_ref[...] = -2.0 * cb
        cn_ref[...] = jnp.sum(cb * cb, axis=1, keepdims=True)
        iota_ref[...] = jax.lax.broadcasted_iota(
            jnp.int32, (_K, _N), 0).astype(jnp.float32)
        lacc_ref[...] = jnp.zeros_like(lacc_ref)
        counts_ref[...] = jnp.zeros_like(counts_ref)

    x = x_ref[0]
    scores2 = jax.lax.dot_general(
        cbn2_ref[...], x, (((1,), (0,)), ((), ())),
        preferred_element_type=jnp.float32)
    dist = cn_ref[...] + scores2

    minv = jnp.min(dist, axis=0, keepdims=True)
    iota_f = iota_ref[...]
    masked = jnp.where(dist == minv, iota_f, float(_K))
    idx = jnp.min(masked, axis=0, keepdims=True)
    oh = (iota_f == idx).astype(jnp.float32)
    oh_ref[0] = oh

    q = jax.lax.dot_general(
        cb, oh, (((0,), (0,)), ((), ())), preferred_element_type=jnp.float32)
    q_ref[0] = q

    diff = q - x
    lacc_ref[...] += diff * diff
    part = oh[:, 0:128]
    for j in range(1, _N // 128):
        part = part + oh[:, 128 * j:128 * (j + 1)]
    counts_ref[...] += part

    @pl.when(i == _B - 1)
    def _finalize():
        loss_ref[0, 0] = jnp.sum(lacc_ref[...]) * (0.25 / (_B * _N * _D))
        p = jnp.sum(counts_ref[...], axis=1, keepdims=True) * (1.0 / (_B * _N))
        ent = jnp.sum(p * jnp.log(p + 1e-10))
        ppl_ref[0, 0] = jnp.exp(-ent)


def _vq_call(inputs, codebook, interpret=False):
    return pl.pallas_call(
        _vq_body,
        grid=(_B,),
        in_specs=[
            pl.BlockSpec((1, _D, _N), lambda i: (i, 0, 0)),
            pl.BlockSpec((_K, _D), lambda i: (0, 0)),
        ],
        out_specs=[
            pl.BlockSpec(memory_space=pltpu.SMEM),
            pl.BlockSpec((1, _D, _N), lambda i: (i, 0, 0)),
            pl.BlockSpec(memory_space=pltpu.SMEM),
            pl.BlockSpec((1, _K, _N), lambda i: (i, 0, 0)),
        ],
        out_shape=[
            jax.ShapeDtypeStruct((1, 1), jnp.float32),
            jax.ShapeDtypeStruct((_B, _D, _N), jnp.float32),
            jax.ShapeDtypeStruct((1, 1), jnp.float32),
            jax.ShapeDtypeStruct((_B, _K, _N), jnp.float32),
        ],
        scratch_shapes=[
            pltpu.VMEM((_K, _D), jnp.float32),
            pltpu.VMEM((_K, 1), jnp.float32),
            pltpu.VMEM((_K, _N), jnp.float32),
            pltpu.VMEM((_D, _N), jnp.float32),
            pltpu.VMEM((_K, 128), jnp.float32),
        ],
        compiler_params=pltpu.CompilerParams(
            dimension_semantics=("arbitrary",)),
        interpret=interpret,
    )(inputs, codebook)


def kernel(inputs, codebook):
    loss, q, ppl, oh = _vq_call(inputs, codebook)
    return (loss[0, 0], q, ppl[0, 0], oh)
